# baseline (speedup 1.0000x reference)
import functools

import jax
import jax.numpy as jnp
from jax.experimental import pallas as pl
from jax.experimental.pallas import tpu as pltpu

TOKENS = 8192
IN_DIM = 1024
HIDDEN = 2048
EXPERTS = 16
TM = 1024
N_SPLIT = 4
CHUNK = TM // N_SPLIT
GROUPS = 8
KG = HIDDEN // GROUPS


def _epilogue(logits):
    iota = jax.lax.broadcasted_iota(jnp.int32, logits.shape, 1)
    m1 = jnp.max(logits, axis=-1, keepdims=True)
    i1 = jnp.argmax(logits, axis=-1, keepdims=True).astype(jnp.int32)
    masked = jnp.where(iota == i1, -jnp.inf, logits)
    m2 = jnp.max(masked, axis=-1, keepdims=True)
    i2 = jnp.argmax(masked, axis=-1, keepdims=True).astype(jnp.int32)

    e = jnp.exp(m2 - m1)
    denom = 1.0 + e
    rout = jnp.where(
        iota == i1, 1.0 / denom, jnp.where(iota == i2, e / denom, 0.0))

    t = jnp.exp((logits - m1) * 100.0)
    ori = t / jnp.sum(t, axis=-1, keepdims=True)

    idx = jnp.concatenate([i1, i2], axis=-1)
    return ori, rout, idx


def _logits_chunk(x, w1_ref, b1_ref, w2_ref, b2_ref):
    h = jnp.dot(x, w1_ref[...], preferred_element_type=jnp.float32)
    h = jnp.maximum(h + b1_ref[...], 0.0)
    logits = jnp.dot(h, w2_ref[...], preferred_element_type=jnp.float32)
    return logits + b2_ref[...]


def _router_block(x_ref, w1_ref, b1_ref, w2_ref, b2_ref,
                  ori_ref, rout_ref, idx_ref):
    chunks = [
        _logits_chunk(x_ref[c * CHUNK:(c + 1) * CHUNK],
                      w1_ref, b1_ref, w2_ref, b2_ref)
        for c in range(N_SPLIT)
    ]
    for c, logits in enumerate(chunks):
        ori, rout, idx = _epilogue(logits)
        sl = slice(c * CHUNK, (c + 1) * CHUNK)
        ori_ref[sl] = ori
        rout_ref[sl] = rout
        idx_ref[sl] = idx


@functools.partial(jax.jit, static_argnames=())
def kernel(x, W1, b1, W2, b2):
    b1r = b1.reshape(1, HIDDEN)
    b2r = b2.reshape(1, EXPERTS)
    grid = (TOKENS // TM,)
    ori, rout, idx = pl.pallas_call(
        _router_block,
        grid=grid,
        in_specs=[
            pl.BlockSpec((TM, IN_DIM), lambda i: (i, 0)),
            pl.BlockSpec((IN_DIM, HIDDEN), lambda i: (0, 0)),
            pl.BlockSpec((1, HIDDEN), lambda i: (0, 0)),
            pl.BlockSpec((HIDDEN, EXPERTS), lambda i: (0, 0)),
            pl.BlockSpec((1, EXPERTS), lambda i: (0, 0)),
        ],
        out_specs=[
            pl.BlockSpec((TM, EXPERTS), lambda i: (i, 0)),
            pl.BlockSpec((TM, EXPERTS), lambda i: (i, 0)),
            pl.BlockSpec((TM, 2), lambda i: (i, 0)),
        ],
        out_shape=[
            jax.ShapeDtypeStruct((TOKENS, EXPERTS), jnp.float32),
            jax.ShapeDtypeStruct((TOKENS, EXPERTS), jnp.float32),
            jax.ShapeDtypeStruct((TOKENS, 2), jnp.int32),
        ],
        compiler_params=pltpu.CompilerParams(
            dimension_semantics=("arbitrary",),
        ),
    )(x, W1, b1r, W2, b2r)
    return (ori, rout, idx)


# speedup vs baseline: 1.0071x; 1.0071x over previous
import functools

import jax
import jax.numpy as jnp
from jax.experimental import pallas as pl
from jax.experimental.pallas import tpu as pltpu

TOKENS = 8192
IN_DIM = 1024
HIDDEN = 2048
EXPERTS = 16
TM = 2048
N_SPLIT = 8
CHUNK = TM // N_SPLIT
GROUPS = 8
KG = HIDDEN // GROUPS


def _epilogue(logits):
    iota = jax.lax.broadcasted_iota(jnp.int32, logits.shape, 1)
    m1 = jnp.max(logits, axis=-1, keepdims=True)
    i1 = jnp.argmax(logits, axis=-1, keepdims=True).astype(jnp.int32)
    masked = jnp.where(iota == i1, -jnp.inf, logits)
    m2 = jnp.max(masked, axis=-1, keepdims=True)
    i2 = jnp.argmax(masked, axis=-1, keepdims=True).astype(jnp.int32)

    e = jnp.exp(m2 - m1)
    denom = 1.0 + e
    rout = jnp.where(
        iota == i1, 1.0 / denom, jnp.where(iota == i2, e / denom, 0.0))

    t = jnp.exp((logits - m1) * 100.0)
    ori = t / jnp.sum(t, axis=-1, keepdims=True)

    idx = jnp.concatenate([i1, i2], axis=-1)
    return ori, rout, idx


def _logits_chunk(x, w1_ref, b1_ref, w2_ref, b2_ref):
    h = jnp.dot(x, w1_ref[...], preferred_element_type=jnp.float32)
    h = jnp.maximum(h + b1_ref[...], 0.0)
    logits = jnp.dot(h, w2_ref[...], preferred_element_type=jnp.float32)
    return logits + b2_ref[...]


def _router_block(x_ref, w1_ref, b1_ref, w2_ref, b2_ref,
                  ori_ref, rout_ref, idx_ref):
    chunks = [
        _logits_chunk(x_ref[c * CHUNK:(c + 1) * CHUNK],
                      w1_ref, b1_ref, w2_ref, b2_ref)
        for c in range(N_SPLIT)
    ]
    for c, logits in enumerate(chunks):
        ori, rout, idx = _epilogue(logits)
        sl = slice(c * CHUNK, (c + 1) * CHUNK)
        ori_ref[sl] = ori
        rout_ref[sl] = rout
        idx_ref[sl] = idx


@functools.partial(jax.jit, static_argnames=())
def kernel(x, W1, b1, W2, b2):
    b1r = b1.reshape(1, HIDDEN)
    b2r = b2.reshape(1, EXPERTS)
    grid = (TOKENS // TM,)
    ori, rout, idx = pl.pallas_call(
        _router_block,
        grid=grid,
        in_specs=[
            pl.BlockSpec((TM, IN_DIM), lambda i: (i, 0)),
            pl.BlockSpec((IN_DIM, HIDDEN), lambda i: (0, 0)),
            pl.BlockSpec((1, HIDDEN), lambda i: (0, 0)),
            pl.BlockSpec((HIDDEN, EXPERTS), lambda i: (0, 0)),
            pl.BlockSpec((1, EXPERTS), lambda i: (0, 0)),
        ],
        out_specs=[
            pl.BlockSpec((TM, EXPERTS), lambda i: (i, 0)),
            pl.BlockSpec((TM, EXPERTS), lambda i: (i, 0)),
            pl.BlockSpec((TM, 2), lambda i: (i, 0)),
        ],
        out_shape=[
            jax.ShapeDtypeStruct((TOKENS, EXPERTS), jnp.float32),
            jax.ShapeDtypeStruct((TOKENS, EXPERTS), jnp.float32),
            jax.ShapeDtypeStruct((TOKENS, 2), jnp.int32),
        ],
        compiler_params=pltpu.CompilerParams(
            dimension_semantics=("parallel",),
        ),
    )(x, W1, b1r, W2, b2r)
    return (ori, rout, idx)


# speedup vs baseline: 1.0126x; 1.0055x over previous
import functools

import jax
import jax.numpy as jnp
from jax.experimental import pallas as pl
from jax.experimental.pallas import tpu as pltpu

TOKENS = 8192
IN_DIM = 1024
HIDDEN = 2048
EXPERTS = 16
TM = 2048
N_SPLIT = 8
CHUNK = TM // N_SPLIT
GROUPS = 8
KG = HIDDEN // GROUPS


def _epilogue(logits):
    iota = jax.lax.broadcasted_iota(jnp.int32, logits.shape, 1)
    m1 = jnp.max(logits, axis=-1, keepdims=True)
    i1 = jnp.argmax(logits, axis=-1, keepdims=True).astype(jnp.int32)
    masked = jnp.where(iota == i1, -jnp.inf, logits)
    m2 = jnp.max(masked, axis=-1, keepdims=True)
    i2 = jnp.argmax(masked, axis=-1, keepdims=True).astype(jnp.int32)

    e = jnp.exp(m2 - m1)
    denom = 1.0 + e
    rout = jnp.where(
        iota == i1, 1.0 / denom, jnp.where(iota == i2, e / denom, 0.0))

    t = jnp.exp((logits - m1) * 100.0)
    ori = t / jnp.sum(t, axis=-1, keepdims=True)

    idx = jnp.concatenate([i1, i2], axis=-1)
    return ori, rout, idx


def _logits_chunk(x, w1_ref, b1_ref, w2_ref, b2_ref):
    h = jnp.dot(x, w1_ref[...], preferred_element_type=jnp.float32)
    h = jnp.maximum(h + b1_ref[...], 0.0)
    logits = jnp.dot(h, w2_ref[...], preferred_element_type=jnp.float32)
    return logits + b2_ref[...]


KSPLIT = 2
KC = IN_DIM // KSPLIT


def _router_block(x_ref, w1_ref, b1_ref, w2_ref, b2_ref,
                  ori_ref, rout_ref, idx_ref, h_ref):
    k = pl.program_id(1)

    @pl.when(k == 0)
    def _first_k_half():
        for c in range(N_SPLIT):
            sl = slice(c * CHUNK, (c + 1) * CHUNK)
            h_ref[sl] = jnp.dot(x_ref[sl], w1_ref[...],
                                preferred_element_type=jnp.float32)

    @pl.when(k == 1)
    def _second_k_half_and_epilogue():
        chunks = []
        for c in range(N_SPLIT):
            sl = slice(c * CHUNK, (c + 1) * CHUNK)
            partial = jnp.dot(x_ref[sl], w1_ref[...],
                              preferred_element_type=jnp.float32)
            h = jnp.maximum(h_ref[sl] + partial + b1_ref[...], 0.0)
            chunks.append(
                jnp.dot(h, w2_ref[...], preferred_element_type=jnp.float32)
                + b2_ref[...])
        for c, logits in enumerate(chunks):
            ori, rout, idx = _epilogue(logits)
            sl = slice(c * CHUNK, (c + 1) * CHUNK)
            ori_ref[sl] = ori
            rout_ref[sl] = rout
            idx_ref[sl] = idx


@functools.partial(jax.jit, static_argnames=())
def kernel(x, W1, b1, W2, b2):
    b1r = b1.reshape(1, HIDDEN)
    b2r = b2.reshape(1, EXPERTS)
    grid = (TOKENS // TM, KSPLIT)
    ori, rout, idx = pl.pallas_call(
        _router_block,
        grid=grid,
        in_specs=[
            pl.BlockSpec((TM, KC), lambda i, k: (i, k)),
            pl.BlockSpec((KC, HIDDEN), lambda i, k: (k, 0)),
            pl.BlockSpec((1, HIDDEN), lambda i, k: (0, 0)),
            pl.BlockSpec((HIDDEN, EXPERTS), lambda i, k: (0, 0)),
            pl.BlockSpec((1, EXPERTS), lambda i, k: (0, 0)),
        ],
        out_specs=[
            pl.BlockSpec((TM, EXPERTS), lambda i, k: (i, 0)),
            pl.BlockSpec((TM, EXPERTS), lambda i, k: (i, 0)),
            pl.BlockSpec((TM, 2), lambda i, k: (i, 0)),
        ],
        out_shape=[
            jax.ShapeDtypeStruct((TOKENS, EXPERTS), jnp.float32),
            jax.ShapeDtypeStruct((TOKENS, EXPERTS), jnp.float32),
            jax.ShapeDtypeStruct((TOKENS, 2), jnp.int32),
        ],
        scratch_shapes=[pltpu.VMEM((TM, HIDDEN), jnp.float32)],
        compiler_params=pltpu.CompilerParams(
            dimension_semantics=("arbitrary", "arbitrary"),
        ),
    )(x, W1, b1r, W2, b2r)
    return (ori, rout, idx)


# speedup vs baseline: 1.0140x; 1.0013x over previous
import functools

import jax
import jax.numpy as jnp
from jax.experimental import pallas as pl
from jax.experimental.pallas import tpu as pltpu

TOKENS = 8192
IN_DIM = 1024
HIDDEN = 2048
EXPERTS = 16
TM = 2048
N_SPLIT = 8
CHUNK = TM // N_SPLIT


def _epilogue(logits):
    iota = jax.lax.broadcasted_iota(jnp.int32, logits.shape, 1)
    m1 = jnp.max(logits, axis=-1, keepdims=True)
    i1 = jnp.argmax(logits, axis=-1, keepdims=True).astype(jnp.int32)
    masked = jnp.where(iota == i1, -jnp.inf, logits)
    m2 = jnp.max(masked, axis=-1, keepdims=True)
    i2 = jnp.argmax(masked, axis=-1, keepdims=True).astype(jnp.int32)

    e = jnp.exp(m2 - m1)
    denom = 1.0 + e
    rout = jnp.where(
        iota == i1, 1.0 / denom, jnp.where(iota == i2, e / denom, 0.0))

    t = jnp.exp((logits - m1) * 100.0)
    ori = t / jnp.sum(t, axis=-1, keepdims=True)

    idx = jnp.concatenate([i1, i2], axis=-1)
    return ori, rout, idx


KSPLIT = 2
KC = IN_DIM // KSPLIT


def _router_block(x_ref, w1_ref, b1_ref, w2_ref, b2_ref,
                  ori_ref, rout_ref, idx_ref, h_ref):
    k = pl.program_id(1)

    @pl.when(k == 0)
    def _first_k_half():
        for c in range(N_SPLIT):
            sl = slice(c * CHUNK, (c + 1) * CHUNK)
            h_ref[sl] = jnp.dot(x_ref[sl], w1_ref[...],
                                preferred_element_type=jnp.float32)

    @pl.when(k == 1)
    def _second_k_half_and_epilogue():
        chunks = []
        for c in range(N_SPLIT):
            sl = slice(c * CHUNK, (c + 1) * CHUNK)
            partial = jnp.dot(x_ref[sl], w1_ref[...],
                              preferred_element_type=jnp.float32)
            h = jnp.maximum(h_ref[sl] + partial + b1_ref[...], 0.0)
            chunks.append(
                jnp.dot(h, w2_ref[...], preferred_element_type=jnp.float32)
                + b2_ref[...])
        for c, logits in enumerate(chunks):
            ori, rout, idx = _epilogue(logits)
            sl = slice(c * CHUNK, (c + 1) * CHUNK)
            ori_ref[sl] = ori
            rout_ref[sl] = rout
            idx_ref[sl] = idx


@functools.partial(jax.jit, static_argnames=())
def kernel(x, W1, b1, W2, b2):
    b1r = b1.reshape(1, HIDDEN)
    b2r = b2.reshape(1, EXPERTS)
    grid = (TOKENS // TM, KSPLIT)
    ori, rout, idx = pl.pallas_call(
        _router_block,
        grid=grid,
        in_specs=[
            pl.BlockSpec((TM, KC), lambda i, k: (i, k)),
            pl.BlockSpec((KC, HIDDEN), lambda i, k: (k, 0)),
            pl.BlockSpec((1, HIDDEN), lambda i, k: (0, 0)),
            pl.BlockSpec((HIDDEN, EXPERTS), lambda i, k: (0, 0)),
            pl.BlockSpec((1, EXPERTS), lambda i, k: (0, 0)),
        ],
        out_specs=[
            pl.BlockSpec((TM, EXPERTS), lambda i, k: (i, 0)),
            pl.BlockSpec((TM, EXPERTS), lambda i, k: (i, 0)),
            pl.BlockSpec((TM, 2), lambda i, k: (i, 0)),
        ],
        out_shape=[
            jax.ShapeDtypeStruct((TOKENS, EXPERTS), jnp.float32),
            jax.ShapeDtypeStruct((TOKENS, EXPERTS), jnp.float32),
            jax.ShapeDtypeStruct((TOKENS, 2), jnp.int32),
        ],
        scratch_shapes=[pltpu.VMEM((TM, HIDDEN), jnp.float32)],
        compiler_params=pltpu.CompilerParams(
            dimension_semantics=("arbitrary", "arbitrary"),
        ),
    )(x, W1, b1r, W2, b2r)
    return (ori, rout, idx)
